# baseline (speedup 1.0000x reference)
import jax
import jax.numpy as jnp
import numpy as np
from jax import lax
from jax.experimental import pallas as pl
from jax.experimental.pallas import tpu as pltpu

_N = 20000
_G = 128
_B = 1024
_NP = 20480
_NB = _NP // _B
_PI = float(np.pi)
_MATCHED = 0.6
_UNMATCHED = 0.45


def _near_bbox(x, y, w, l, r):
    rot = r - jnp.floor(r / _PI + 0.5) * _PI
    cond = jnp.abs(rot) > (_PI / 4.0)
    dx = jnp.where(cond, l, w)
    dy = jnp.where(cond, w, l)
    return x - dx / 2.0, y - dy / 2.0, x + dx / 2.0, y + dy / 2.0


def _assign_body(a_ref, gt_ref, gtall_ref, bbox_ref, lab_ref, rw_ref,
                 rmax_ref, rarg_ref):
    gx = gt_ref[:, 0:1]
    gy = gt_ref[:, 1:2]
    gw = gt_ref[:, 3:4]
    gl = gt_ref[:, 4:5]
    gr = gt_ref[:, 6:7]
    gxmin, gymin, gxmax, gymax = _near_bbox(gx, gy, gw, gl, gr)
    garea = (gxmax - gxmin) * (gymax - gymin)
    gtall = gtall_ref[:, :]

    subiota = lax.broadcasted_iota(jnp.int32, (_G, _B), 0)
    laneiota = lax.broadcasted_iota(jnp.int32, (_G, _B), 1)
    lane1 = lax.broadcasted_iota(jnp.int32, (1, _B), 1)

    def p1(i, carry):
        cmax, carg = carry
        ax = a_ref[i, 0:1, :]
        ay = a_ref[i, 1:2, :]
        aw = a_ref[i, 3:4, :]
        al = a_ref[i, 4:5, :]
        ar = a_ref[i, 6:7, :]
        axmin, aymin, axmax, aymax = _near_bbox(ax, ay, aw, al, ar)
        aarea = (axmax - axmin) * (aymax - aymin)
        iw = jnp.clip(jnp.minimum(axmax, gxmax) - jnp.maximum(axmin, gxmin),
                      0.0, None)
        ih = jnp.clip(jnp.minimum(aymax, gymax) - jnp.maximum(aymin, gymin),
                      0.0, None)
        inter = iw * ih
        union = aarea + garea - inter
        iou = inter / jnp.maximum(union, 1e-8)

        rmax = jnp.max(iou, axis=0, keepdims=True)
        rarg = jnp.min(jnp.where(iou == rmax, subiota, _G),
                       axis=0, keepdims=True)
        rmax_ref[i] = rmax
        rarg_ref[i] = rarg

        bcmax = jnp.max(iou, axis=1, keepdims=True)
        bcarg = jnp.min(jnp.where(iou == bcmax, laneiota, _NP),
                        axis=1, keepdims=True) + i * _B
        upd = bcmax > cmax
        return jnp.where(upd, bcmax, cmax), jnp.where(upd, bcarg, carg)

    cmax0 = jnp.full((_G, 1), -1.0, jnp.float32)
    carg0 = jnp.zeros((_G, 1), jnp.int32)
    _, carg = lax.fori_loop(0, _NB, p1, (cmax0, carg0))

    def p2(i, _):
        glane = lane1 + i * _B
        eq = carg == glane
        bestj = jnp.max(jnp.where(eq, subiota, -1),
                        axis=0, keepdims=True)
        forced = bestj >= 0
        rmax = rmax_ref[i]
        rarg = rarg_ref[i]
        farg = jnp.where(forced, bestj, rarg)
        onehot = (subiota == farg).astype(jnp.float32)

        sel = jnp.sum(gtall[:, :, None] * onehot[None, :, :], axis=1)
        xg = sel[0:1, :]
        yg = sel[1:2, :]
        zg = sel[2:3, :]
        wg = sel[3:4, :]
        lg = sel[4:5, :]
        hg = sel[5:6, :]
        rg = sel[6:7, :]
        clsv = sel[7:8, :]

        base = jnp.where(rmax < _UNMATCHED, 0.0, -1.0)
        base = jnp.where(rmax >= _MATCHED, clsv, base)
        labf = jnp.where(forced, clsv, base)
        labi = labf.astype(jnp.int32)
        fgf = (labi > 0).astype(jnp.float32)

        xa = a_ref[i, 0:1, :]
        ya = a_ref[i, 1:2, :]
        za = a_ref[i, 2:3, :]
        wa = a_ref[i, 3:4, :]
        la = a_ref[i, 4:5, :]
        ha = a_ref[i, 5:6, :]
        ra = a_ref[i, 6:7, :]
        diag = jnp.sqrt(la * la + wa * wa)
        bbox_ref[i, 0:1, :] = (xg - xa) / diag * fgf
        bbox_ref[i, 1:2, :] = (yg - ya) / diag * fgf
        bbox_ref[i, 2:3, :] = (zg - za) / ha * fgf
        bbox_ref[i, 3:4, :] = jnp.log(wg / wa) * fgf
        bbox_ref[i, 4:5, :] = jnp.log(lg / la) * fgf
        bbox_ref[i, 5:6, :] = jnp.log(hg / ha) * fgf
        bbox_ref[i, 6:7, :] = (rg - ra) * fgf
        lab_ref[i] = labi
        rw_ref[i] = fgf
        return 0

    lax.fori_loop(0, _NB, p2, 0)


@jax.jit
def kernel(anchors, gt_boxes, gt_classes):
    npad = _NP - _N
    pad = jnp.concatenate(
        [jnp.full((npad, 3), 1e9, jnp.float32),
         jnp.ones((npad, 3), jnp.float32),
         jnp.zeros((npad, 1), jnp.float32)], axis=1)
    a_pad = jnp.concatenate([anchors.astype(jnp.float32), pad], axis=0)
    a3 = a_pad.T.reshape(7, _NB, _B).transpose(1, 0, 2)
    gtall = jnp.concatenate(
        [gt_boxes.astype(jnp.float32).T,
         gt_classes.astype(jnp.float32)[None, :]], axis=0)

    bbox3, lab3, rw3 = pl.pallas_call(
        _assign_body,
        out_shape=[
            jax.ShapeDtypeStruct((_NB, 7, _B), jnp.float32),
            jax.ShapeDtypeStruct((_NB, 1, _B), jnp.int32),
            jax.ShapeDtypeStruct((_NB, 1, _B), jnp.float32),
        ],
        scratch_shapes=[
            pltpu.VMEM((_NB, 1, _B), jnp.float32),
            pltpu.VMEM((_NB, 1, _B), jnp.int32),
        ],
    )(a3, gt_boxes.astype(jnp.float32), gtall)

    bbox_targets = bbox3.transpose(0, 2, 1).reshape(_NP, 7)[:_N]
    labels = lab3.reshape(_NP)[:_N]
    reg_weights = rw3.reshape(_NP)[:_N]
    return bbox_targets, labels, reg_weights


# speedup vs baseline: 1.0957x; 1.0957x over previous
import jax
import jax.numpy as jnp
import numpy as np
from jax import lax
from jax.experimental import pallas as pl
from jax.experimental.pallas import tpu as pltpu

_N = 20000
_G = 128
_B = 1024
_NP = 20480
_NB = _NP // _B
_PI = float(np.pi)
_MATCHED = 0.6
_UNMATCHED = 0.45


def _near_bbox(x, y, w, l, r):
    rot = r - jnp.floor(r / _PI + 0.5) * _PI
    cond = jnp.abs(rot) > (_PI / 4.0)
    dx = jnp.where(cond, l, w)
    dy = jnp.where(cond, w, l)
    return x - dx / 2.0, y - dy / 2.0, x + dx / 2.0, y + dy / 2.0


def _assign_body(a_ref, gt_ref, gtall_ref, bbox_ref, lab_ref, rw_ref,
                 rmax_ref, rarg_ref):
    gx = gt_ref[:, 0:1]
    gy = gt_ref[:, 1:2]
    gw = gt_ref[:, 3:4]
    gl = gt_ref[:, 4:5]
    gr = gt_ref[:, 6:7]
    gxmin, gymin, gxmax, gymax = _near_bbox(gx, gy, gw, gl, gr)
    garea = (gxmax - gxmin) * (gymax - gymin)
    gtall = [gtall_ref[k] for k in range(3)]

    subiota = lax.broadcasted_iota(jnp.int32, (_G, _B), 0)
    laneiota = lax.broadcasted_iota(jnp.int32, (_G, _B), 1)
    lane1 = lax.broadcasted_iota(jnp.int32, (1, _B), 1)

    def p1(i, carry):
        cmax, carg = carry
        ax = a_ref[i, 0:1, :]
        ay = a_ref[i, 1:2, :]
        aw = a_ref[i, 3:4, :]
        al = a_ref[i, 4:5, :]
        ar = a_ref[i, 6:7, :]
        axmin, aymin, axmax, aymax = _near_bbox(ax, ay, aw, al, ar)
        aarea = (axmax - axmin) * (aymax - aymin)
        iw = jnp.clip(jnp.minimum(axmax, gxmax) - jnp.maximum(axmin, gxmin),
                      0.0, None)
        ih = jnp.clip(jnp.minimum(aymax, gymax) - jnp.maximum(aymin, gymin),
                      0.0, None)
        inter = iw * ih
        union = aarea + garea - inter
        iou = inter / jnp.maximum(union, 1e-8)

        rmax = jnp.max(iou, axis=0, keepdims=True)
        rarg = jnp.min(jnp.where(iou == rmax, subiota, _G),
                       axis=0, keepdims=True)
        rmax_ref[i] = rmax
        rarg_ref[i] = rarg

        bcmax = jnp.max(iou, axis=1, keepdims=True)
        bcarg = jnp.min(jnp.where(iou == bcmax, laneiota, _NP),
                        axis=1, keepdims=True) + i * _B
        upd = bcmax > cmax
        return jnp.where(upd, bcmax, cmax), jnp.where(upd, bcarg, carg)

    cmax0 = jnp.full((_G, 1), -1.0, jnp.float32)
    carg0 = jnp.zeros((_G, 1), jnp.int32)
    _, carg = lax.fori_loop(0, _NB, p1, (cmax0, carg0))

    def p2(i, _):
        glane = lane1 + i * _B
        eq = carg == glane
        bestj = jnp.max(jnp.where(eq, subiota, -1),
                        axis=0, keepdims=True)
        forced = bestj >= 0
        rmax = rmax_ref[i]
        rarg = rarg_ref[i]
        farg = jnp.where(forced, bestj, rarg)
        onehot = (subiota == farg).astype(jnp.bfloat16)

        def dot(g):
            return lax.dot_general(g, onehot, (((1,), (0,)), ((), ())),
                                   preferred_element_type=jnp.float32)
        sel = (dot(gtall[0]) + dot(gtall[1])) + dot(gtall[2])
        xg = sel[0:1, :]
        yg = sel[1:2, :]
        zg = sel[2:3, :]
        wg = sel[3:4, :]
        lg = sel[4:5, :]
        hg = sel[5:6, :]
        rg = sel[6:7, :]
        clsv = sel[7:8, :]

        base = jnp.where(rmax < _UNMATCHED, 0.0, -1.0)
        base = jnp.where(rmax >= _MATCHED, clsv, base)
        labf = jnp.where(forced, clsv, base)
        labi = labf.astype(jnp.int32)
        fgf = (labi > 0).astype(jnp.float32)

        xa = a_ref[i, 0:1, :]
        ya = a_ref[i, 1:2, :]
        za = a_ref[i, 2:3, :]
        wa = a_ref[i, 3:4, :]
        la = a_ref[i, 4:5, :]
        ha = a_ref[i, 5:6, :]
        ra = a_ref[i, 6:7, :]
        diag = jnp.sqrt(la * la + wa * wa)
        bbox_ref[i, 0:1, :] = (xg - xa) / diag * fgf
        bbox_ref[i, 1:2, :] = (yg - ya) / diag * fgf
        bbox_ref[i, 2:3, :] = (zg - za) / ha * fgf
        bbox_ref[i, 3:4, :] = jnp.log(wg / wa) * fgf
        bbox_ref[i, 4:5, :] = jnp.log(lg / la) * fgf
        bbox_ref[i, 5:6, :] = jnp.log(hg / ha) * fgf
        bbox_ref[i, 6:7, :] = (rg - ra) * fgf
        lab_ref[i] = labi
        rw_ref[i] = fgf
        return 0

    lax.fori_loop(0, _NB, p2, 0)


@jax.jit
def kernel(anchors, gt_boxes, gt_classes):
    npad = _NP - _N
    pad = jnp.concatenate(
        [jnp.full((npad, 3), 1e9, jnp.float32),
         jnp.ones((npad, 3), jnp.float32),
         jnp.zeros((npad, 1), jnp.float32)], axis=1)
    a_pad = jnp.concatenate([anchors.astype(jnp.float32), pad], axis=0)
    a3 = a_pad.T.reshape(7, _NB, _B).transpose(1, 0, 2)
    gtall = jnp.concatenate(
        [gt_boxes.astype(jnp.float32).T,
         gt_classes.astype(jnp.float32)[None, :]], axis=0)
    t0 = gtall.astype(jnp.bfloat16)
    r1 = gtall - t0.astype(jnp.float32)
    t1 = r1.astype(jnp.bfloat16)
    t2 = (r1 - t1.astype(jnp.float32)).astype(jnp.bfloat16)
    gt3 = jnp.stack([t0, t1, t2], axis=0)

    bbox3, lab3, rw3 = pl.pallas_call(
        _assign_body,
        out_shape=[
            jax.ShapeDtypeStruct((_NB, 7, _B), jnp.float32),
            jax.ShapeDtypeStruct((_NB, 1, _B), jnp.int32),
            jax.ShapeDtypeStruct((_NB, 1, _B), jnp.float32),
        ],
        scratch_shapes=[
            pltpu.VMEM((_NB, 1, _B), jnp.float32),
            pltpu.VMEM((_NB, 1, _B), jnp.int32),
        ],
    )(a3, gt_boxes.astype(jnp.float32), gt3)

    bbox_targets = bbox3.transpose(0, 2, 1).reshape(_NP, 7)[:_N]
    labels = lab3.reshape(_NP)[:_N]
    reg_weights = rw3.reshape(_NP)[:_N]
    return bbox_targets, labels, reg_weights


# speedup vs baseline: 1.0971x; 1.0013x over previous
import jax
import jax.numpy as jnp
import numpy as np
from jax import lax
from jax.experimental import pallas as pl
from jax.experimental.pallas import tpu as pltpu

_N = 20000
_G = 128
_B = 1024
_NP = 20480
_NB = _NP // _B
_PI = float(np.pi)
_MATCHED = 0.6
_UNMATCHED = 0.45


def _near_bbox(x, y, w, l, r):
    rot = r - jnp.floor(r / _PI + 0.5) * _PI
    cond = jnp.abs(rot) > (_PI / 4.0)
    dx = jnp.where(cond, l, w)
    dy = jnp.where(cond, w, l)
    return x - dx / 2.0, y - dy / 2.0, x + dx / 2.0, y + dy / 2.0


def _assign_body(a_ref, gt_ref, gtall_ref, bbox_ref, lab_ref, rw_ref,
                 rmax_ref, rarg_ref):
    gx = gt_ref[:, 0:1]
    gy = gt_ref[:, 1:2]
    gw = gt_ref[:, 3:4]
    gl = gt_ref[:, 4:5]
    gr = gt_ref[:, 6:7]
    gxmin, gymin, gxmax, gymax = _near_bbox(gx, gy, gw, gl, gr)
    garea = (gxmax - gxmin) * (gymax - gymin)
    gt_f = gtall_ref[...]
    t0 = gt_f.astype(jnp.bfloat16)
    r1 = gt_f - t0.astype(jnp.float32)
    t1 = r1.astype(jnp.bfloat16)
    t2 = (r1 - t1.astype(jnp.float32)).astype(jnp.bfloat16)
    gtall = [t0, t1, t2]

    subiota = lax.broadcasted_iota(jnp.int32, (_G, _B), 0)
    laneiota = lax.broadcasted_iota(jnp.int32, (_G, _B), 1)
    lane1 = lax.broadcasted_iota(jnp.int32, (1, _B), 1)

    def p1(i, carry):
        cmax, carg = carry
        ax = a_ref[i, 0:1, :]
        ay = a_ref[i, 1:2, :]
        aw = a_ref[i, 3:4, :]
        al = a_ref[i, 4:5, :]
        ar = a_ref[i, 6:7, :]
        axmin, aymin, axmax, aymax = _near_bbox(ax, ay, aw, al, ar)
        aarea = (axmax - axmin) * (aymax - aymin)
        iw = jnp.clip(jnp.minimum(axmax, gxmax) - jnp.maximum(axmin, gxmin),
                      0.0, None)
        ih = jnp.clip(jnp.minimum(aymax, gymax) - jnp.maximum(aymin, gymin),
                      0.0, None)
        inter = iw * ih
        union = aarea + garea - inter
        iou = inter / jnp.maximum(union, 1e-8)

        rmax = jnp.max(iou, axis=0, keepdims=True)
        rarg = jnp.min(jnp.where(iou == rmax, subiota, _G),
                       axis=0, keepdims=True)
        rmax_ref[i] = rmax
        rarg_ref[i] = rarg

        bcmax = jnp.max(iou, axis=1, keepdims=True)
        bcarg = jnp.min(jnp.where(iou == bcmax, laneiota, _NP),
                        axis=1, keepdims=True) + i * _B
        upd = bcmax > cmax
        return jnp.where(upd, bcmax, cmax), jnp.where(upd, bcarg, carg)

    cmax0 = jnp.full((_G, 1), -1.0, jnp.float32)
    carg0 = jnp.zeros((_G, 1), jnp.int32)
    _, carg = lax.fori_loop(0, _NB, p1, (cmax0, carg0))

    def p2(i, _):
        glane = lane1 + i * _B
        eq = carg == glane
        bestj = jnp.max(jnp.where(eq, subiota, -1),
                        axis=0, keepdims=True)
        forced = bestj >= 0
        rmax = rmax_ref[i]
        rarg = rarg_ref[i]
        farg = jnp.where(forced, bestj, rarg)
        onehot = (subiota == farg).astype(jnp.bfloat16)

        def dot(g):
            return lax.dot_general(g, onehot, (((1,), (0,)), ((), ())),
                                   preferred_element_type=jnp.float32)
        sel = (dot(gtall[0]) + dot(gtall[1])) + dot(gtall[2])
        xg = sel[0:1, :]
        yg = sel[1:2, :]
        zg = sel[2:3, :]
        wg = sel[3:4, :]
        lg = sel[4:5, :]
        hg = sel[5:6, :]
        rg = sel[6:7, :]
        clsv = sel[7:8, :]

        base = jnp.where(rmax < _UNMATCHED, 0.0, -1.0)
        base = jnp.where(rmax >= _MATCHED, clsv, base)
        labf = jnp.where(forced, clsv, base)
        labi = labf.astype(jnp.int32)
        fgf = (labi > 0).astype(jnp.float32)

        xa = a_ref[i, 0:1, :]
        ya = a_ref[i, 1:2, :]
        za = a_ref[i, 2:3, :]
        wa = a_ref[i, 3:4, :]
        la = a_ref[i, 4:5, :]
        ha = a_ref[i, 5:6, :]
        ra = a_ref[i, 6:7, :]
        diag = jnp.sqrt(la * la + wa * wa)
        bbox_ref[i, 0:1, :] = (xg - xa) / diag * fgf
        bbox_ref[i, 1:2, :] = (yg - ya) / diag * fgf
        bbox_ref[i, 2:3, :] = (zg - za) / ha * fgf
        bbox_ref[i, 3:4, :] = jnp.log(wg / wa) * fgf
        bbox_ref[i, 4:5, :] = jnp.log(lg / la) * fgf
        bbox_ref[i, 5:6, :] = jnp.log(hg / ha) * fgf
        bbox_ref[i, 6:7, :] = (rg - ra) * fgf
        lab_ref[i] = labi
        rw_ref[i] = fgf
        return 0

    lax.fori_loop(0, _NB, p2, 0)


@jax.jit
def kernel(anchors, gt_boxes, gt_classes):
    npad = _NP - _N
    pad = jnp.concatenate(
        [jnp.full((npad, 3), 1e9, jnp.float32),
         jnp.ones((npad, 3), jnp.float32),
         jnp.zeros((npad, 1), jnp.float32)], axis=1)
    a_pad = jnp.concatenate([anchors.astype(jnp.float32), pad], axis=0)
    a3 = a_pad.T.reshape(7, _NB, _B).transpose(1, 0, 2)
    gt3 = jnp.concatenate(
        [gt_boxes.astype(jnp.float32).T,
         gt_classes.astype(jnp.float32)[None, :]], axis=0)

    bbox3, lab3, rw3 = pl.pallas_call(
        _assign_body,
        out_shape=[
            jax.ShapeDtypeStruct((_NB, 7, _B), jnp.float32),
            jax.ShapeDtypeStruct((_NB, 1, _B), jnp.int32),
            jax.ShapeDtypeStruct((_NB, 1, _B), jnp.float32),
        ],
        scratch_shapes=[
            pltpu.VMEM((_NB, 1, _B), jnp.float32),
            pltpu.VMEM((_NB, 1, _B), jnp.int32),
        ],
    )(a3, gt_boxes.astype(jnp.float32), gt3)

    bbox_targets = bbox3.transpose(0, 2, 1).reshape(_NP, 7)[:_N]
    labels = lab3.reshape(_NP)[:_N]
    reg_weights = rw3.reshape(_NP)[:_N]
    return bbox_targets, labels, reg_weights


# speedup vs baseline: 1.2490x; 1.1386x over previous
import jax
import jax.numpy as jnp
import numpy as np
from jax import lax
from jax.experimental import pallas as pl
from jax.experimental.pallas import tpu as pltpu

_N = 20000
_G = 128
_B = 2048
_NP = 20480
_NB = _NP // _B
_PI = float(np.pi)
_MATCHED = 0.6
_UNMATCHED = 0.45


def _near_bbox(x, y, w, l, r):
    rot = r - jnp.floor(r / _PI + 0.5) * _PI
    cond = jnp.abs(rot) > (_PI / 4.0)
    dx = jnp.where(cond, l, w)
    dy = jnp.where(cond, w, l)
    return x - dx / 2.0, y - dy / 2.0, x + dx / 2.0, y + dy / 2.0


def _assign_body(a_ref, gt_ref, gtall_ref, bbox_ref, lab_ref, rw_ref,
                 rmax_ref, rarg_ref):
    gx = gt_ref[:, 0:1]
    gy = gt_ref[:, 1:2]
    gw = gt_ref[:, 3:4]
    gl = gt_ref[:, 4:5]
    gr = gt_ref[:, 6:7]
    gxmin, gymin, gxmax, gymax = _near_bbox(gx, gy, gw, gl, gr)
    garea = (gxmax - gxmin) * (gymax - gymin)
    gt_f = gtall_ref[...]
    t0 = gt_f.astype(jnp.bfloat16)
    r1 = gt_f - t0.astype(jnp.float32)
    t1 = r1.astype(jnp.bfloat16)
    t2 = (r1 - t1.astype(jnp.float32)).astype(jnp.bfloat16)
    gtall = [t0, t1, t2]

    subiota = lax.broadcasted_iota(jnp.int32, (_G, _B), 0)
    laneiota = lax.broadcasted_iota(jnp.int32, (_G, _B), 1)
    lane1 = lax.broadcasted_iota(jnp.int32, (1, _B), 1)

    def p1(i, carry):
        cmax, carg = carry
        ax = a_ref[i, 0:1, :]
        ay = a_ref[i, 1:2, :]
        aw = a_ref[i, 3:4, :]
        al = a_ref[i, 4:5, :]
        ar = a_ref[i, 6:7, :]
        axmin, aymin, axmax, aymax = _near_bbox(ax, ay, aw, al, ar)
        aarea = (axmax - axmin) * (aymax - aymin)
        iw = jnp.clip(jnp.minimum(axmax, gxmax) - jnp.maximum(axmin, gxmin),
                      0.0, None)
        ih = jnp.clip(jnp.minimum(aymax, gymax) - jnp.maximum(aymin, gymin),
                      0.0, None)
        inter = iw * ih
        union = aarea + garea - inter
        iou = inter / jnp.maximum(union, 1e-8)

        rmax = jnp.max(iou, axis=0, keepdims=True)
        rarg = jnp.min(jnp.where(iou == rmax, subiota, _G),
                       axis=0, keepdims=True)
        rmax_ref[i] = rmax
        rarg_ref[i] = rarg

        bcmax = jnp.max(iou, axis=1, keepdims=True)
        bcarg = jnp.min(jnp.where(iou == bcmax, laneiota, _NP),
                        axis=1, keepdims=True) + i * _B
        upd = bcmax > cmax
        return jnp.where(upd, bcmax, cmax), jnp.where(upd, bcarg, carg)

    cmax0 = jnp.full((_G, 1), -1.0, jnp.float32)
    carg0 = jnp.zeros((_G, 1), jnp.int32)
    _, carg = lax.fori_loop(0, _NB, p1, (cmax0, carg0))

    def p2(i, _):
        glane = lane1 + i * _B
        eq = carg == glane
        bestj = jnp.max(jnp.where(eq, subiota, -1),
                        axis=0, keepdims=True)
        forced = bestj >= 0
        rmax = rmax_ref[i]
        rarg = rarg_ref[i]
        farg = jnp.where(forced, bestj, rarg)
        onehot = (subiota == farg).astype(jnp.bfloat16)

        def dot(g):
            return lax.dot_general(g, onehot, (((1,), (0,)), ((), ())),
                                   preferred_element_type=jnp.float32)
        sel = (dot(gtall[0]) + dot(gtall[1])) + dot(gtall[2])
        xg = sel[0:1, :]
        yg = sel[1:2, :]
        zg = sel[2:3, :]
        wg = sel[3:4, :]
        lg = sel[4:5, :]
        hg = sel[5:6, :]
        rg = sel[6:7, :]
        clsv = sel[7:8, :]

        base = jnp.where(rmax < _UNMATCHED, 0.0, -1.0)
        base = jnp.where(rmax >= _MATCHED, clsv, base)
        labf = jnp.where(forced, clsv, base)
        labi = labf.astype(jnp.int32)
        fgf = (labi > 0).astype(jnp.float32)

        xa = a_ref[i, 0:1, :]
        ya = a_ref[i, 1:2, :]
        za = a_ref[i, 2:3, :]
        wa = a_ref[i, 3:4, :]
        la = a_ref[i, 4:5, :]
        ha = a_ref[i, 5:6, :]
        ra = a_ref[i, 6:7, :]
        diag = jnp.sqrt(la * la + wa * wa)
        bbox_ref[i, 0:1, :] = (xg - xa) / diag * fgf
        bbox_ref[i, 1:2, :] = (yg - ya) / diag * fgf
        bbox_ref[i, 2:3, :] = (zg - za) / ha * fgf
        bbox_ref[i, 3:4, :] = jnp.log(wg / wa) * fgf
        bbox_ref[i, 4:5, :] = jnp.log(lg / la) * fgf
        bbox_ref[i, 5:6, :] = jnp.log(hg / ha) * fgf
        bbox_ref[i, 6:7, :] = (rg - ra) * fgf
        lab_ref[i] = labi
        rw_ref[i] = fgf
        return 0

    lax.fori_loop(0, _NB, p2, 0)


@jax.jit
def kernel(anchors, gt_boxes, gt_classes):
    npad = _NP - _N
    pad = jnp.concatenate(
        [jnp.full((npad, 3), 1e9, jnp.float32),
         jnp.ones((npad, 3), jnp.float32),
         jnp.zeros((npad, 1), jnp.float32)], axis=1)
    a_pad = jnp.concatenate([anchors.astype(jnp.float32), pad], axis=0)
    a3 = a_pad.T.reshape(7, _NB, _B).transpose(1, 0, 2)
    gt3 = jnp.concatenate(
        [gt_boxes.astype(jnp.float32).T,
         gt_classes.astype(jnp.float32)[None, :]], axis=0)

    bbox3, lab3, rw3 = pl.pallas_call(
        _assign_body,
        out_shape=[
            jax.ShapeDtypeStruct((_NB, 7, _B), jnp.float32),
            jax.ShapeDtypeStruct((_NB, 1, _B), jnp.int32),
            jax.ShapeDtypeStruct((_NB, 1, _B), jnp.float32),
        ],
        scratch_shapes=[
            pltpu.VMEM((_NB, 1, _B), jnp.float32),
            pltpu.VMEM((_NB, 1, _B), jnp.int32),
        ],
    )(a3, gt_boxes.astype(jnp.float32), gt3)

    bbox_targets = bbox3.transpose(0, 2, 1).reshape(_NP, 7)[:_N]
    labels = lab3.reshape(_NP)[:_N]
    reg_weights = rw3.reshape(_NP)[:_N]
    return bbox_targets, labels, reg_weights


# speedup vs baseline: 1.3103x; 1.0490x over previous
import jax
import jax.numpy as jnp
import numpy as np
from jax import lax
from jax.experimental import pallas as pl
from jax.experimental.pallas import tpu as pltpu

_N = 20000
_G = 128
_B = 4096
_NP = 20480
_NB = _NP // _B
_PI = float(np.pi)
_MATCHED = 0.6
_UNMATCHED = 0.45


def _near_bbox(x, y, w, l, r):
    rot = r - jnp.floor(r / _PI + 0.5) * _PI
    cond = jnp.abs(rot) > (_PI / 4.0)
    dx = jnp.where(cond, l, w)
    dy = jnp.where(cond, w, l)
    return x - dx / 2.0, y - dy / 2.0, x + dx / 2.0, y + dy / 2.0


def _assign_body(a_ref, gt_ref, gtall_ref, bbox_ref, lab_ref, rw_ref,
                 rmax_ref, rarg_ref):
    gx = gt_ref[:, 0:1]
    gy = gt_ref[:, 1:2]
    gw = gt_ref[:, 3:4]
    gl = gt_ref[:, 4:5]
    gr = gt_ref[:, 6:7]
    gxmin, gymin, gxmax, gymax = _near_bbox(gx, gy, gw, gl, gr)
    garea = (gxmax - gxmin) * (gymax - gymin)
    gt_f = gtall_ref[...]
    t0 = gt_f.astype(jnp.bfloat16)
    r1 = gt_f - t0.astype(jnp.float32)
    t1 = r1.astype(jnp.bfloat16)
    t2 = (r1 - t1.astype(jnp.float32)).astype(jnp.bfloat16)
    gtall = [t0, t1, t2]

    subiota = lax.broadcasted_iota(jnp.int32, (_G, _B), 0)
    laneiota = lax.broadcasted_iota(jnp.int32, (_G, _B), 1)
    lane1 = lax.broadcasted_iota(jnp.int32, (1, _B), 1)

    def p1(i, carry):
        cmax, carg = carry
        ax = a_ref[i, 0:1, :]
        ay = a_ref[i, 1:2, :]
        aw = a_ref[i, 3:4, :]
        al = a_ref[i, 4:5, :]
        ar = a_ref[i, 6:7, :]
        axmin, aymin, axmax, aymax = _near_bbox(ax, ay, aw, al, ar)
        aarea = (axmax - axmin) * (aymax - aymin)
        iw = jnp.clip(jnp.minimum(axmax, gxmax) - jnp.maximum(axmin, gxmin),
                      0.0, None)
        ih = jnp.clip(jnp.minimum(aymax, gymax) - jnp.maximum(aymin, gymin),
                      0.0, None)
        inter = iw * ih
        union = aarea + garea - inter
        iou = inter / jnp.maximum(union, 1e-8)

        rmax = jnp.max(iou, axis=0, keepdims=True)
        rarg = jnp.min(jnp.where(iou == rmax, subiota, _G),
                       axis=0, keepdims=True)
        rmax_ref[i] = rmax
        rarg_ref[i] = rarg

        bcmax = jnp.max(iou, axis=1, keepdims=True)
        bcarg = jnp.min(jnp.where(iou == bcmax, laneiota, _NP),
                        axis=1, keepdims=True) + i * _B
        upd = bcmax > cmax
        return jnp.where(upd, bcmax, cmax), jnp.where(upd, bcarg, carg)

    cmax0 = jnp.full((_G, 1), -1.0, jnp.float32)
    carg0 = jnp.zeros((_G, 1), jnp.int32)
    _, carg = lax.fori_loop(0, _NB, p1, (cmax0, carg0))

    def p2(i, _):
        glane = lane1 + i * _B
        eq = carg == glane
        bestj = jnp.max(jnp.where(eq, subiota, -1),
                        axis=0, keepdims=True)
        forced = bestj >= 0
        rmax = rmax_ref[i]
        rarg = rarg_ref[i]
        farg = jnp.where(forced, bestj, rarg)
        onehot = (subiota == farg).astype(jnp.bfloat16)

        def dot(g):
            return lax.dot_general(g, onehot, (((1,), (0,)), ((), ())),
                                   preferred_element_type=jnp.float32)
        sel = (dot(gtall[0]) + dot(gtall[1])) + dot(gtall[2])
        xg = sel[0:1, :]
        yg = sel[1:2, :]
        zg = sel[2:3, :]
        wg = sel[3:4, :]
        lg = sel[4:5, :]
        hg = sel[5:6, :]
        rg = sel[6:7, :]
        clsv = sel[7:8, :]

        base = jnp.where(rmax < _UNMATCHED, 0.0, -1.0)
        base = jnp.where(rmax >= _MATCHED, clsv, base)
        labf = jnp.where(forced, clsv, base)
        labi = labf.astype(jnp.int32)
        fgf = (labi > 0).astype(jnp.float32)

        xa = a_ref[i, 0:1, :]
        ya = a_ref[i, 1:2, :]
        za = a_ref[i, 2:3, :]
        wa = a_ref[i, 3:4, :]
        la = a_ref[i, 4:5, :]
        ha = a_ref[i, 5:6, :]
        ra = a_ref[i, 6:7, :]
        diag = jnp.sqrt(la * la + wa * wa)
        bbox_ref[i, 0:1, :] = (xg - xa) / diag * fgf
        bbox_ref[i, 1:2, :] = (yg - ya) / diag * fgf
        bbox_ref[i, 2:3, :] = (zg - za) / ha * fgf
        bbox_ref[i, 3:4, :] = jnp.log(wg / wa) * fgf
        bbox_ref[i, 4:5, :] = jnp.log(lg / la) * fgf
        bbox_ref[i, 5:6, :] = jnp.log(hg / ha) * fgf
        bbox_ref[i, 6:7, :] = (rg - ra) * fgf
        lab_ref[i] = labi
        rw_ref[i] = fgf
        return 0

    lax.fori_loop(0, _NB, p2, 0)


@jax.jit
def kernel(anchors, gt_boxes, gt_classes):
    npad = _NP - _N
    pad = jnp.concatenate(
        [jnp.full((npad, 3), 1e9, jnp.float32),
         jnp.ones((npad, 3), jnp.float32),
         jnp.zeros((npad, 1), jnp.float32)], axis=1)
    a_pad = jnp.concatenate([anchors.astype(jnp.float32), pad], axis=0)
    a3 = a_pad.T.reshape(7, _NB, _B).transpose(1, 0, 2)
    gt3 = jnp.concatenate(
        [gt_boxes.astype(jnp.float32).T,
         gt_classes.astype(jnp.float32)[None, :]], axis=0)

    bbox3, lab3, rw3 = pl.pallas_call(
        _assign_body,
        out_shape=[
            jax.ShapeDtypeStruct((_NB, 7, _B), jnp.float32),
            jax.ShapeDtypeStruct((_NB, 1, _B), jnp.int32),
            jax.ShapeDtypeStruct((_NB, 1, _B), jnp.float32),
        ],
        scratch_shapes=[
            pltpu.VMEM((_NB, 1, _B), jnp.float32),
            pltpu.VMEM((_NB, 1, _B), jnp.int32),
        ],
    )(a3, gt_boxes.astype(jnp.float32), gt3)

    bbox_targets = bbox3.transpose(0, 2, 1).reshape(_NP, 7)[:_N]
    labels = lab3.reshape(_NP)[:_N]
    reg_weights = rw3.reshape(_NP)[:_N]
    return bbox_targets, labels, reg_weights


# speedup vs baseline: 1.5471x; 1.1807x over previous
import jax
import jax.numpy as jnp
import numpy as np
from jax import lax
from jax.experimental import pallas as pl
from jax.experimental.pallas import tpu as pltpu

_N = 20000
_G = 128
_B = 4096
_PI = float(np.pi)
_MATCHED = 0.6
_UNMATCHED = 0.45
_BLOCKS = [(o, min(_B, _N - o)) for o in range(0, _N, _B)]


def _near_bbox(x, y, w, l, r):
    rot = r - jnp.floor(r / _PI + 0.5) * _PI
    cond = jnp.abs(rot) > (_PI / 4.0)
    dx = jnp.where(cond, l, w)
    dy = jnp.where(cond, w, l)
    return x - dx / 2.0, y - dy / 2.0, x + dx / 2.0, y + dy / 2.0


def _assign_body(a_ref, gt_ref, gtall_ref, bbox_ref, lab_ref, rw_ref,
                 rmax_ref, rarg_ref):
    gx = gt_ref[:, 0:1]
    gy = gt_ref[:, 1:2]
    gw = gt_ref[:, 3:4]
    gl = gt_ref[:, 4:5]
    gr = gt_ref[:, 6:7]
    gxmin, gymin, gxmax, gymax = _near_bbox(gx, gy, gw, gl, gr)
    garea = (gxmax - gxmin) * (gymax - gymin)
    gt_f = gtall_ref[...]
    t0 = gt_f.astype(jnp.bfloat16)
    r1 = gt_f - t0.astype(jnp.float32)
    t1 = r1.astype(jnp.bfloat16)
    t2 = (r1 - t1.astype(jnp.float32)).astype(jnp.bfloat16)

    cmax = jnp.full((_G, 1), -1.0, jnp.float32)
    carg = jnp.zeros((_G, 1), jnp.int32)
    for off, w_ in _BLOCKS:
        subiota = lax.broadcasted_iota(jnp.int32, (_G, w_), 0)
        laneiota = lax.broadcasted_iota(jnp.int32, (_G, w_), 1)
        ax = a_ref[0:1, off:off + w_]
        ay = a_ref[1:2, off:off + w_]
        aw = a_ref[3:4, off:off + w_]
        al = a_ref[4:5, off:off + w_]
        ar = a_ref[6:7, off:off + w_]
        axmin, aymin, axmax, aymax = _near_bbox(ax, ay, aw, al, ar)
        aarea = (axmax - axmin) * (aymax - aymin)
        iw = jnp.clip(jnp.minimum(axmax, gxmax) - jnp.maximum(axmin, gxmin),
                      0.0, None)
        ih = jnp.clip(jnp.minimum(aymax, gymax) - jnp.maximum(aymin, gymin),
                      0.0, None)
        inter = iw * ih
        union = aarea + garea - inter
        iou = inter / jnp.maximum(union, 1e-8)

        rmax = jnp.max(iou, axis=0, keepdims=True)
        rarg = jnp.min(jnp.where(iou == rmax, subiota, _G),
                       axis=0, keepdims=True)
        rmax_ref[0:1, off:off + w_] = rmax
        rarg_ref[0:1, off:off + w_] = rarg

        bcmax = jnp.max(iou, axis=1, keepdims=True)
        bcarg = jnp.min(jnp.where(iou == bcmax, laneiota, _N),
                        axis=1, keepdims=True) + off
        upd = bcmax > cmax
        cmax = jnp.where(upd, bcmax, cmax)
        carg = jnp.where(upd, bcarg, carg)

    for off, w_ in _BLOCKS:
        subiota = lax.broadcasted_iota(jnp.int32, (_G, w_), 0)
        glane = lax.broadcasted_iota(jnp.int32, (1, w_), 1) + off
        eq = carg == glane
        bestj = jnp.max(jnp.where(eq, subiota, -1),
                        axis=0, keepdims=True)
        forced = bestj >= 0
        rmax = rmax_ref[0:1, off:off + w_]
        rarg = rarg_ref[0:1, off:off + w_]
        farg = jnp.where(forced, bestj, rarg)
        onehot = (subiota == farg).astype(jnp.bfloat16)

        def dot(g):
            return lax.dot_general(g, onehot, (((1,), (0,)), ((), ())),
                                   preferred_element_type=jnp.float32)
        sel = (dot(t0) + dot(t1)) + dot(t2)
        xg = sel[0:1, :]
        yg = sel[1:2, :]
        zg = sel[2:3, :]
        wg = sel[3:4, :]
        lg = sel[4:5, :]
        hg = sel[5:6, :]
        rg = sel[6:7, :]
        clsv = sel[7:8, :]

        base = jnp.where(rmax < _UNMATCHED, 0.0, -1.0)
        base = jnp.where(rmax >= _MATCHED, clsv, base)
        labf = jnp.where(forced, clsv, base)
        labi = labf.astype(jnp.int32)
        fgf = (labi > 0).astype(jnp.float32)

        xa = a_ref[0:1, off:off + w_]
        ya = a_ref[1:2, off:off + w_]
        za = a_ref[2:3, off:off + w_]
        wa = a_ref[3:4, off:off + w_]
        la = a_ref[4:5, off:off + w_]
        ha = a_ref[5:6, off:off + w_]
        ra = a_ref[6:7, off:off + w_]
        diag = jnp.sqrt(la * la + wa * wa)
        bbox_ref[0:1, off:off + w_] = (xg - xa) / diag * fgf
        bbox_ref[1:2, off:off + w_] = (yg - ya) / diag * fgf
        bbox_ref[2:3, off:off + w_] = (zg - za) / ha * fgf
        bbox_ref[3:4, off:off + w_] = jnp.log(wg / wa) * fgf
        bbox_ref[4:5, off:off + w_] = jnp.log(lg / la) * fgf
        bbox_ref[5:6, off:off + w_] = jnp.log(hg / ha) * fgf
        bbox_ref[6:7, off:off + w_] = (rg - ra) * fgf
        lab_ref[0:1, off:off + w_] = labi
        rw_ref[0:1, off:off + w_] = fgf


@jax.jit
def kernel(anchors, gt_boxes, gt_classes):
    a_t = anchors.astype(jnp.float32).T
    gt3 = jnp.concatenate(
        [gt_boxes.astype(jnp.float32).T,
         gt_classes.astype(jnp.float32)[None, :]], axis=0)

    bbox_t, lab2, rw2 = pl.pallas_call(
        _assign_body,
        out_shape=[
            jax.ShapeDtypeStruct((7, _N), jnp.float32),
            jax.ShapeDtypeStruct((1, _N), jnp.int32),
            jax.ShapeDtypeStruct((1, _N), jnp.float32),
        ],
        scratch_shapes=[
            pltpu.VMEM((1, _N), jnp.float32),
            pltpu.VMEM((1, _N), jnp.int32),
        ],
    )(a_t, gt_boxes.astype(jnp.float32), gt3)

    return bbox_t.T, lab2.reshape(_N), rw2.reshape(_N)


# speedup vs baseline: 1.5567x; 1.0062x over previous
import jax
import jax.numpy as jnp
import numpy as np
from jax import lax
from jax.experimental import pallas as pl
from jax.experimental.pallas import tpu as pltpu

_N = 20000
_G = 128
_B = 4096
_PI = float(np.pi)
_MATCHED = 0.6
_UNMATCHED = 0.45
_BLOCKS = [(o, min(_B, _N - o)) for o in range(0, _N, _B)]


def _near_bbox(x, y, w, l, r):
    rot = r - jnp.floor(r / _PI + 0.5) * _PI
    cond = jnp.abs(rot) > (_PI / 4.0)
    dx = jnp.where(cond, l, w)
    dy = jnp.where(cond, w, l)
    return x - dx / 2.0, y - dy / 2.0, x + dx / 2.0, y + dy / 2.0


def _assign_body(a_ref, gt_ref, gtall_ref, bbox_ref, lab_ref, rw_ref):
    gx = gt_ref[:, 0:1]
    gy = gt_ref[:, 1:2]
    gw = gt_ref[:, 3:4]
    gl = gt_ref[:, 4:5]
    gr = gt_ref[:, 6:7]
    gxmin, gymin, gxmax, gymax = _near_bbox(gx, gy, gw, gl, gr)
    garea = (gxmax - gxmin) * (gymax - gymin)
    gt_f = gtall_ref[...]
    t0 = gt_f.astype(jnp.bfloat16)
    r1 = gt_f - t0.astype(jnp.float32)
    t1 = r1.astype(jnp.bfloat16)
    t2 = (r1 - t1.astype(jnp.float32)).astype(jnp.bfloat16)

    cmax = jnp.full((_G, 1), -1.0, jnp.float32)
    carg = jnp.zeros((_G, 1), jnp.int32)
    rmaxs = []
    rargs = []
    for off, w_ in _BLOCKS:
        subiota = lax.broadcasted_iota(jnp.int32, (_G, w_), 0)
        laneiota = lax.broadcasted_iota(jnp.int32, (_G, w_), 1)
        ax = a_ref[0:1, off:off + w_]
        ay = a_ref[1:2, off:off + w_]
        aw = a_ref[3:4, off:off + w_]
        al = a_ref[4:5, off:off + w_]
        ar = a_ref[6:7, off:off + w_]
        axmin, aymin, axmax, aymax = _near_bbox(ax, ay, aw, al, ar)
        aarea = (axmax - axmin) * (aymax - aymin)
        iw = jnp.clip(jnp.minimum(axmax, gxmax) - jnp.maximum(axmin, gxmin),
                      0.0, None)
        ih = jnp.clip(jnp.minimum(aymax, gymax) - jnp.maximum(aymin, gymin),
                      0.0, None)
        inter = iw * ih
        union = aarea + garea - inter
        iou = inter / jnp.maximum(union, 1e-8)

        rmax = jnp.max(iou, axis=0, keepdims=True)
        rarg = jnp.min(jnp.where(iou == rmax, subiota, _G),
                       axis=0, keepdims=True)
        rmaxs.append(rmax)
        rargs.append(rarg)

        bcmax = jnp.max(iou, axis=1, keepdims=True)
        bcarg = jnp.min(jnp.where(iou == bcmax, laneiota, _N),
                        axis=1, keepdims=True) + off
        upd = bcmax > cmax
        cmax = jnp.where(upd, bcmax, cmax)
        carg = jnp.where(upd, bcarg, carg)

    for bi, (off, w_) in enumerate(_BLOCKS):
        subiota = lax.broadcasted_iota(jnp.int32, (_G, w_), 0)
        glane = lax.broadcasted_iota(jnp.int32, (1, w_), 1) + off
        eq = carg == glane
        bestj = jnp.max(jnp.where(eq, subiota, -1),
                        axis=0, keepdims=True)
        forced = bestj >= 0
        rmax = rmaxs[bi]
        rarg = rargs[bi]
        farg = jnp.where(forced, bestj, rarg)
        onehot = (subiota == farg).astype(jnp.bfloat16)

        def dot(g):
            return lax.dot_general(g, onehot, (((1,), (0,)), ((), ())),
                                   preferred_element_type=jnp.float32)
        sel = (dot(t0) + dot(t1)) + dot(t2)
        xg = sel[0:1, :]
        yg = sel[1:2, :]
        zg = sel[2:3, :]
        wg = sel[3:4, :]
        lg = sel[4:5, :]
        hg = sel[5:6, :]
        rg = sel[6:7, :]
        clsv = sel[7:8, :]

        base = jnp.where(rmax < _UNMATCHED, 0.0, -1.0)
        base = jnp.where(rmax >= _MATCHED, clsv, base)
        labf = jnp.where(forced, clsv, base)
        labi = labf.astype(jnp.int32)
        fgf = (labi > 0).astype(jnp.float32)

        xa = a_ref[0:1, off:off + w_]
        ya = a_ref[1:2, off:off + w_]
        za = a_ref[2:3, off:off + w_]
        wa = a_ref[3:4, off:off + w_]
        la = a_ref[4:5, off:off + w_]
        ha = a_ref[5:6, off:off + w_]
        ra = a_ref[6:7, off:off + w_]
        diag = jnp.sqrt(la * la + wa * wa)
        bbox_ref[0:1, off:off + w_] = (xg - xa) / diag * fgf
        bbox_ref[1:2, off:off + w_] = (yg - ya) / diag * fgf
        bbox_ref[2:3, off:off + w_] = (zg - za) / ha * fgf
        bbox_ref[3:4, off:off + w_] = jnp.log(wg / wa) * fgf
        bbox_ref[4:5, off:off + w_] = jnp.log(lg / la) * fgf
        bbox_ref[5:6, off:off + w_] = jnp.log(hg / ha) * fgf
        bbox_ref[6:7, off:off + w_] = (rg - ra) * fgf
        lab_ref[0:1, off:off + w_] = labi
        rw_ref[0:1, off:off + w_] = fgf


@jax.jit
def kernel(anchors, gt_boxes, gt_classes):
    a_t = anchors.astype(jnp.float32).T
    gt3 = jnp.concatenate(
        [gt_boxes.astype(jnp.float32).T,
         gt_classes.astype(jnp.float32)[None, :]], axis=0)

    bbox_t, lab2, rw2 = pl.pallas_call(
        _assign_body,
        out_shape=[
            jax.ShapeDtypeStruct((7, _N), jnp.float32),
            jax.ShapeDtypeStruct((1, _N), jnp.int32),
            jax.ShapeDtypeStruct((1, _N), jnp.float32),
        ],
    )(a_t, gt_boxes.astype(jnp.float32), gt3)

    return bbox_t.T, lab2.reshape(_N), rw2.reshape(_N)
